# baseline (speedup 1.0000x reference)
import functools

import jax
import jax.numpy as jnp
from jax import lax
from jax.experimental import pallas as pl
from jax.experimental.pallas import tpu as pltpu
from jax.experimental.pallas import tpu_sc as plsc

B = 16384
D = 512
N_MIX = 65536
LANES = 16
NC = 2
NS = 16
NW = NC * NS
ROWS_PER_W = N_MIX // NW
K = 16
NT = ROWS_PER_W // K
NBUF = 4

_mesh = plsc.VectorSubcoreMesh(
    core_axis_name="c", subcore_axis_name="s", num_cores=NC, num_subcores=NS
)


@functools.partial(
    pl.kernel,
    out_type=jax.ShapeDtypeStruct((N_MIX, D), jnp.float32),
    mesh=_mesh,
    compiler_params=pltpu.CompilerParams(
        needs_layout_passes=False, use_tc_tiling_on_sc=True
    ),
    scratch_types=[
        pltpu.VMEM((ROWS_PER_W,), jnp.int32),
        pltpu.VMEM((ROWS_PER_W,), jnp.int32),
        pltpu.VMEM((ROWS_PER_W,), jnp.float32),
        [pltpu.VMEM((K, D), jnp.float32)] * NBUF,
        [pltpu.VMEM((K, D), jnp.float32)] * NBUF,
        [pltpu.VMEM((K, D), jnp.float32)] * NBUF,
        [pltpu.SemaphoreType.DMA] * NBUF,
        [pltpu.SemaphoreType.DMA] * NBUF,
        [pltpu.SemaphoreType.DMA] * NBUF,
    ],
)
def _mix_sc(x_hbm, ia_hbm, ib_hbm, lam_hbm, out_hbm,
            ia_v, ib_v, lam_v, xa, xb, ob, sa, sb, sw):
    wid = lax.axis_index("s") * NC + lax.axis_index("c")

    pltpu.sync_copy(ia_hbm.at[wid], ia_v)
    pltpu.sync_copy(ib_hbm.at[wid], ib_v)
    pltpu.sync_copy(lam_hbm.at[wid], lam_v)

    row0 = wid * ROWS_PER_W

    def issue_gathers(t, buf):
        pltpu.async_copy(x_hbm.at[ia_v.at[pl.ds(t * K, K)]], xa[buf], sa[buf])
        pltpu.async_copy(x_hbm.at[ib_v.at[pl.ds(t * K, K)]], xb[buf], sb[buf])

    for t in range(NBUF - 1):
        issue_gathers(t, t)

    def quad_body(q, _):
        for buf in range(NBUF):
            t = NBUF * q + buf
            pltpu.make_async_copy(
                x_hbm.at[ia_v.at[pl.ds(t * K, K)]], xa[buf], sa[buf]
            ).wait()
            pltpu.make_async_copy(
                x_hbm.at[ib_v.at[pl.ds(t * K, K)]], xb[buf], sb[buf]
            ).wait()

            @pl.when(q > 0)
            def _():
                pltpu.make_async_copy(
                    ob[buf], out_hbm.at[pl.ds(row0, K)], sw[buf]
                ).wait()

            xa_b, xb_b, o_b = xa[buf], xb[buf], ob[buf]

            def row_body(r, _):
                lam16 = plsc.load_gather(
                    lam_v, [jnp.full((LANES,), t * K + r, jnp.int32)]
                )
                for c in range(D // LANES):
                    sl = pl.ds(c * LANES, LANES)
                    av = xa_b[r, sl]
                    bv = xb_b[r, sl]
                    o_b[r, sl] = bv + lam16 * (av - bv)
                return _

            lax.fori_loop(0, K, row_body, None)

            pltpu.async_copy(o_b, out_hbm.at[pl.ds(row0 + t * K, K)], sw[buf])

            @pl.when(t + NBUF - 1 < NT)
            def _():
                issue_gathers(t + NBUF - 1, (buf + NBUF - 1) % NBUF)
        return _

    lax.fori_loop(0, NT // NBUF, quad_body, None)

    for buf in range(NBUF):
        t = NT - NBUF + buf
        pltpu.make_async_copy(
            ob[buf], out_hbm.at[pl.ds(row0 + t * K, K)], sw[buf]
        ).wait()


def kernel(x, idx_a, idx_b, mix_lambda):
    ia = idx_a.astype(jnp.int32).reshape(NW, ROWS_PER_W)
    ib = idx_b.astype(jnp.int32).reshape(NW, ROWS_PER_W)
    lam = mix_lambda.astype(jnp.float32).reshape(NW, ROWS_PER_W)
    return _mix_sc(x, ia, ib, lam)
